# baseline (speedup 1.0000x reference)
import functools

import jax
import jax.numpy as jnp
from jax import lax
from jax.experimental import pallas as pl
from jax.experimental.pallas import tpu as pltpu
from jax.experimental.pallas import tpu_sc as plsc

B = 16384
D = 64
MARGIN = 0.1
EPS = 1e-5

NC = 2
NS = 16
NW = NC * NS
RPT = B // NW
NGRP = RPT // 16

NCLS = 9
NREL = 2

_SC_PARAMS = dict(
    mesh=plsc.VectorSubcoreMesh(core_axis_name="c", subcore_axis_name="s"),
    compiler_params=pltpu.CompilerParams(use_tc_tiling_on_sc=False),
)


def _iota16():
    return lax.broadcasted_iota(jnp.int32, (16,), 0)


def _accumulate_stats(buf, g, stats_v):
    def step(r, carry):
        out = []
        outsq = []
        for c in range(4):
            v = buf[r, pl.ds(c * 16, 16)]
            out.append(carry[c] + v)
            outsq.append(carry[4 + c] + v * v)
        return tuple(out) + tuple(outsq)

    init = tuple(jnp.zeros((16,), jnp.float32) for _ in range(8))
    accs = lax.fori_loop(0, RPT, step, init, unroll=8)
    for c in range(4):
        stats_v[g, pl.ds(c * 16, 16)] = accs[c]
        stats_v[NCLS + g, pl.ds(c * 16, 16)] = accs[4 + c]


def _compact_rad(radb, icls, g, rad_cv):
    iota = _iota16()

    def step(grp, _):
        lo16 = lax.bitwise_and(icls[g, pl.ds(grp * 16, 16)], 15)
        cvec = jnp.zeros((16,), jnp.float32)
        for k in range(16):
            row = radb[grp * 16 + k, :]
            sel = row.at[lo16].get(mode="promise_in_bounds")
            cvec = jnp.where(iota == k, sel, cvec)
        rad_cv[pl.ds(grp * 16, 16)] = cvec
        return 0

    lax.fori_loop(0, NGRP, step, 0)


def _sc_gather_body(cls_idx, rel_idx, cls_tab, rad_tab, rel_tab,
                    cls_out, rel_out, rad_out, stats_out,
                    icls, ihi, irel, rowb0, rowb1, radb0, radb1,
                    radc0, radc1, stats_v,
                    isem, gsa0, gsa1, gsb0, gsb1, wsa0, wsa1, wsb0, wsb1):
    wid = lax.axis_index("s") * NC + lax.axis_index("c")
    base = wid * RPT
    rowb = [rowb0, rowb1]
    radb = [radb0, radb1]
    radc = [radc0, radc1]
    gsa = [gsa0, gsa1]
    gsb = [gsb0, gsb1]
    wsa = [wsa0, wsa1]
    wsb = [wsb0, wsb1]

    h1 = pltpu.async_copy(cls_idx.at[:, 0, pl.ds(base, RPT)], icls, isem)
    h2 = pltpu.async_copy(rel_idx.at[:, 0, pl.ds(base, RPT)], irel, isem)
    h1.wait()
    h2.wait()
    for g in range(NCLS):
        for grp in range(NGRP):
            sl = pl.ds(grp * 16, 16)
            ihi[g, sl] = lax.shift_right_logical(icls[g, sl], 4)

    NA = NCLS + NREL
    NB = NCLS

    def issue_a(i):
        b = i % 2
        if i < NCLS:
            return pltpu.async_copy(cls_tab.at[icls.at[i]], rowb[b], gsa[b])
        return pltpu.async_copy(rel_tab.at[irel.at[i - NCLS]], rowb[b], gsa[b])

    def retire_a(i, h):
        b = i % 2
        h.wait()
        if i < NCLS:
            _accumulate_stats(rowb[b], i, stats_v)
            dst = cls_out.at[i, pl.ds(base, RPT)]
        else:
            dst = rel_out.at[i - NCLS, pl.ds(base, RPT)]
        return pltpu.async_copy(rowb[b], dst, wsa[b])

    def issue_b(i):
        b = i % 2
        return pltpu.async_copy(rad_tab.at[ihi.at[i]], radb[b], gsb[b])

    def retire_b(i, h):
        b = i % 2
        h.wait()
        _compact_rad(radb[b], icls, i, radc[b])
        return pltpu.async_copy(radc[b], rad_out.at[i, 0, pl.ds(base, RPT)],
                                wsb[b])

    ha = [None] * NA
    hb = [None] * NB
    wa = [None] * NA
    wb = [None] * NB
    ha[0] = issue_a(0)
    hb[0] = issue_b(0)
    for i in range(NA):
        if i + 1 < NA:
            if i - 1 >= 0:
                wa[i - 1].wait()
            ha[i + 1] = issue_a(i + 1)
        if i + 1 < NB:
            if i - 1 >= 0:
                wb[i - 1].wait()
            hb[i + 1] = issue_b(i + 1)
        wa[i] = retire_a(i, ha[i])
        if i < NB:
            wb[i] = retire_b(i, hb[i])
    wa[NA - 2].wait()
    wa[NA - 1].wait()
    wb[NB - 2].wait()
    wb[NB - 1].wait()
    pltpu.sync_copy(stats_v, stats_out.at[wid])


@functools.cache
def _sc_gather():
    return functools.partial(
        pl.kernel,
        out_type=[
            jax.ShapeDtypeStruct((NCLS, B, D), jnp.float32),
            jax.ShapeDtypeStruct((NREL, B, D), jnp.float32),
            jax.ShapeDtypeStruct((NCLS, 1, B), jnp.float32),
            jax.ShapeDtypeStruct((NW, 2 * NCLS, D), jnp.float32),
        ],
        scratch_types=[
            pltpu.VMEM((NCLS, RPT), jnp.int32),
            pltpu.VMEM((NCLS, RPT), jnp.int32),
            pltpu.VMEM((NREL, RPT), jnp.int32),
            pltpu.VMEM((RPT, D), jnp.float32),
            pltpu.VMEM((RPT, D), jnp.float32),
            pltpu.VMEM((RPT, 16), jnp.float32),
            pltpu.VMEM((RPT, 16), jnp.float32),
            pltpu.VMEM((RPT,), jnp.float32),
            pltpu.VMEM((RPT,), jnp.float32),
            pltpu.VMEM((2 * NCLS, D), jnp.float32),
        ] + [pltpu.SemaphoreType.DMA] * 9,
        **_SC_PARAMS,
    )(_sc_gather_body)


def _rsqrt_nr(x):
    i = lax.bitcast_convert_type(x, jnp.int32)
    i = 0x5F3759DF - lax.shift_right_logical(i, 1)
    y = lax.bitcast_convert_type(i, jnp.float32)
    for _ in range(3):
        y = y * (1.5 - 0.5 * x * y * y)
    return y


def _sqrt_nr(x):
    return x * _rsqrt_nr(x)


def _relu(x):
    return jnp.maximum(x, 0.0)


def _sc_loss_body(cls_rows, rel_rows, rad, scale, shift, out,
                  ba, bb, bc, ss_v, sh_v, rad_v, res_v, sem0, sem1, sem2):
    wid = lax.axis_index("s") * NC + lax.axis_index("c")
    base = wid * RPT
    iota = _iota16()

    pltpu.sync_copy(scale, ss_v)
    pltpu.sync_copy(shift, sh_v)
    pltpu.sync_copy(rad.at[:, 0, pl.ds(base, RPT)], rad_v)

    def fetch(buf, src, g, sem):
        return pltpu.async_copy(src.at[g, pl.ds(base, RPT)], buf, sem)

    def consts(g):
        s = [ss_v[g, pl.ds(c * 16, 16)] for c in range(4)]
        m = [sh_v[g, pl.ds(c * 16, 16)] for c in range(4)]
        return s, m

    rots = [lax.bitwise_and(iota + s, 15) for s in (8, 4, 2, 1)]

    def hsum16(v):
        for rot in rots:
            v = v + v.at[rot].get(mode="promise_in_bounds")
        return v

    def rowsum_merge(rowsums, k, n2):
        return jnp.where(iota == k, hsum16(n2), rowsums)

    def rad_at(g, grp):
        return jnp.abs(rad_v[g, pl.ds(grp * 16, 16)])

    acc = jnp.zeros((16,), jnp.float32)

    h1 = fetch(ba, cls_rows, 0, sem0)
    h2 = fetch(bb, cls_rows, 1, sem1)
    h1.wait()
    h2.wait()
    sa, ma = consts(0)
    sb, mb = consts(1)
    dm = [ma[c] - mb[c] for c in range(4)]

    def g0(grp, acc):
        rs = jnp.zeros((16,), jnp.float32)
        for k in range(16):
            r = grp * 16 + k
            n2 = None
            for c in range(4):
                sl = pl.ds(c * 16, 16)
                d = (ba[r, sl] * sa[c] - bb[r, sl] * sb[c]) - dm[c]
                t = d * d
                n2 = t if n2 is None else n2 + t
            rs = rowsum_merge(rs, k, n2)
        dst = _sqrt_nr(rs)
        return acc + _relu(dst + rad_at(0, grp) - rad_at(1, grp) - MARGIN)

    acc = lax.fori_loop(0, NGRP, g0, acc, unroll=4)

    h1 = fetch(ba, cls_rows, 2, sem0)
    h2 = fetch(bb, cls_rows, 3, sem1)
    h3 = fetch(bc, cls_rows, 4, sem2)
    h1.wait()
    h2.wait()
    h3.wait()
    sa, ma = consts(2)
    sb, mb = consts(3)
    se, me = consts(4)

    def g1(grp, acc):
        rs1 = jnp.zeros((16,), jnp.float32)
        rs2 = jnp.zeros((16,), jnp.float32)
        rs3 = jnp.zeros((16,), jnp.float32)
        for k in range(16):
            r = grp * 16 + k
            n21 = n22 = n23 = None
            for c in range(4):
                sl = pl.ds(c * 16, 16)
                av = ba[r, sl] * sa[c] - ma[c]
                bv = bb[r, sl] * sb[c] - mb[c]
                ev = bc[r, sl] * se[c] - me[c]
                d1 = av - bv
                d2 = ev - av
                d3 = ev - bv
                t1 = d1 * d1
                t2 = d2 * d2
                t3 = d3 * d3
                n21 = t1 if n21 is None else n21 + t1
                n22 = t2 if n22 is None else n22 + t2
                n23 = t3 if n23 is None else n23 + t3
            rs1 = rowsum_merge(rs1, k, n21)
            rs2 = rowsum_merge(rs2, k, n22)
            rs3 = rowsum_merge(rs3, k, n23)
        ra = rad_at(2, grp)
        rb = rad_at(3, grp)
        t = _relu(_sqrt_nr(rs1) - (ra + rb) - MARGIN)
        t += _relu(_sqrt_nr(rs2) - ra - MARGIN)
        t += _relu(_sqrt_nr(rs3) - rb - MARGIN)
        return acc + t

    acc = lax.fori_loop(0, NGRP, g1, acc, unroll=4)

    h1 = fetch(ba, cls_rows, 5, sem0)
    h2 = fetch(bb, cls_rows, 6, sem1)
    h3 = fetch(bc, rel_rows, 0, sem2)
    h1.wait()
    h2.wait()
    h3.wait()
    sa, ma = consts(5)
    sb, mb = consts(6)
    dm = [ma[c] - mb[c] for c in range(4)]

    def g2(grp, acc):
        rs = jnp.zeros((16,), jnp.float32)
        for k in range(16):
            r = grp * 16 + k
            n2 = None
            for c in range(4):
                sl = pl.ds(c * 16, 16)
                d = (ba[r, sl] * sa[c] - bb[r, sl] * sb[c]) + bc[r, sl] - dm[c]
                t = d * d
                n2 = t if n2 is None else n2 + t
            rs = rowsum_merge(rs, k, n2)
        dst = _sqrt_nr(rs)
        ra = rad_at(5, grp)
        rb = rad_at(6, grp)
        t = _relu(dst + ra - rb - MARGIN)
        t += _relu(ra + rb - dst + MARGIN)
        return acc + t

    acc = lax.fori_loop(0, NGRP, g2, acc, unroll=4)

    h1 = fetch(ba, cls_rows, 7, sem0)
    h2 = fetch(bb, cls_rows, 8, sem1)
    h3 = fetch(bc, rel_rows, 1, sem2)
    h1.wait()
    h2.wait()
    h3.wait()
    sa, ma = consts(7)
    sb, mb = consts(8)
    dm = [ma[c] - mb[c] for c in range(4)]

    def g3(grp, acc):
        rs = jnp.zeros((16,), jnp.float32)
        for k in range(16):
            r = grp * 16 + k
            n2 = None
            for c in range(4):
                sl = pl.ds(c * 16, 16)
                d = (ba[r, sl] * sa[c] - bb[r, sl] * sb[c]) - bc[r, sl] - dm[c]
                t = d * d
                n2 = t if n2 is None else n2 + t
            rs = rowsum_merge(rs, k, n2)
        euc = _sqrt_nr(rs)
        ra = rad_at(7, grp)
        rb = rad_at(8, grp)
        return acc + _relu(euc - ra - rb - MARGIN)

    acc = lax.fori_loop(0, NGRP, g3, acc, unroll=4)

    res_v[...] = acc
    pltpu.sync_copy(res_v, out.at[wid])


@functools.cache
def _sc_loss():
    return functools.partial(
        pl.kernel,
        out_type=jax.ShapeDtypeStruct((NW, 16), jnp.float32),
        scratch_types=[
            pltpu.VMEM((RPT, D), jnp.float32),
            pltpu.VMEM((RPT, D), jnp.float32),
            pltpu.VMEM((RPT, D), jnp.float32),
            pltpu.VMEM((NCLS, D), jnp.float32),
            pltpu.VMEM((NCLS, D), jnp.float32),
            pltpu.VMEM((NCLS, RPT), jnp.float32),
            pltpu.VMEM((16,), jnp.float32),
            pltpu.SemaphoreType.DMA,
            pltpu.SemaphoreType.DMA,
            pltpu.SemaphoreType.DMA,
        ],
        **_SC_PARAMS,
    )(_sc_loss_body)


def kernel(nf0, nf1, nf2, nf3, class_embed_w, class_rad_w, rel_embed_w,
           bn_gamma, bn_beta):
    del bn_beta
    cls_idx = jnp.stack([
        nf0[:, 0], nf0[:, 1],
        nf1[:, 0], nf1[:, 1], nf1[:, 2],
        nf2[:, 0], nf2[:, 2],
        nf3[:, 1], nf3[:, 2],
    ]).astype(jnp.int32).reshape(NCLS, 1, B)
    rel_idx = jnp.stack([nf2[:, 1], nf3[:, 0]]).astype(jnp.int32)

    cls_rows, rel_rows, rad, stats = _sc_gather()(
        cls_idx, rel_idx.reshape(NREL, 1, B),
        class_embed_w, class_rad_w.reshape(-1, 16), rel_embed_w)

    sums = jnp.sum(stats[:, :NCLS, :], axis=0)
    sumsqs = jnp.sum(stats[:, NCLS:, :], axis=0)
    inv_b = 1.0 / B
    mean = sums * inv_b
    var = sumsqs * inv_b - mean * mean
    scale = bn_gamma[None, :] / jnp.sqrt(var + EPS)
    shift = scale * mean

    parts = _sc_loss()(cls_rows, rel_rows, rad, scale, shift)
    return jnp.sum(parts) * inv_b


# speedup vs baseline: 1.0747x; 1.0747x over previous
import functools

import jax
import jax.numpy as jnp
from jax import lax
from jax.experimental import pallas as pl
from jax.experimental.pallas import tpu as pltpu
from jax.experimental.pallas import tpu_sc as plsc

B = 16384
D = 64
MARGIN = 0.1
EPS = 1e-5

NC = 2
NS = 16
NW = NC * NS
RPT = B // NW
NGRP = RPT // 16

NCLS = 9
NREL = 2

_SC_PARAMS = dict(
    mesh=plsc.VectorSubcoreMesh(core_axis_name="c", subcore_axis_name="s"),
    compiler_params=pltpu.CompilerParams(use_tc_tiling_on_sc=False),
)


def _iota16():
    return lax.broadcasted_iota(jnp.int32, (16,), 0)


def _accumulate_stats(buf, g, stats_v):
    def step(r, carry):
        out = []
        outsq = []
        for c in range(4):
            v = buf[r, pl.ds(c * 16, 16)]
            out.append(carry[c] + v)
            outsq.append(carry[4 + c] + v * v)
        return tuple(out) + tuple(outsq)

    init = tuple(jnp.zeros((16,), jnp.float32) for _ in range(8))
    accs = lax.fori_loop(0, RPT, step, init, unroll=8)
    for c in range(4):
        stats_v[g, pl.ds(c * 16, 16)] = accs[c]
        stats_v[NCLS + g, pl.ds(c * 16, 16)] = accs[4 + c]


def _compact_rad(radb, icls, g, rad_cv):
    iota = _iota16()

    def step(grp, _):
        lo16 = lax.bitwise_and(icls[g, pl.ds(grp * 16, 16)], 15)
        cvec = jnp.zeros((16,), jnp.float32)
        for k in range(16):
            row = radb[grp * 16 + k, :]
            sel = row.at[lo16].get(mode="promise_in_bounds")
            cvec = jnp.where(iota == k, sel, cvec)
        rad_cv[pl.ds(grp * 16, 16)] = cvec
        return 0

    lax.fori_loop(0, NGRP, step, 0)


def _sc_gather_body(cls_idx, rel_idx, cls_tab, rad_tab, rel_tab,
                    cls_out, rel_out, rad_out, stats_out,
                    icls, ihi, irel, rowb0, rowb1, radb0, radb1,
                    radc0, radc1, stats_v,
                    isem, gsa0, gsa1, gsb0, gsb1, wsa0, wsa1, wsb0, wsb1):
    wid = lax.axis_index("s") * NC + lax.axis_index("c")
    base = wid * RPT
    rowb = [rowb0, rowb1]
    radb = [radb0, radb1]
    radc = [radc0, radc1]
    gsa = [gsa0, gsa1]
    gsb = [gsb0, gsb1]
    wsa = [wsa0, wsa1]
    wsb = [wsb0, wsb1]

    h1 = pltpu.async_copy(cls_idx.at[:, 0, pl.ds(base, RPT)], icls, isem)
    h2 = pltpu.async_copy(rel_idx.at[:, 0, pl.ds(base, RPT)], irel, isem)
    h1.wait()
    h2.wait()
    for g in range(NCLS):
        for grp in range(NGRP):
            sl = pl.ds(grp * 16, 16)
            ihi[g, sl] = lax.shift_right_logical(icls[g, sl], 4)

    NA = NCLS + NREL
    NB = NCLS

    def issue_a(i):
        b = i % 2
        if i < NCLS:
            return pltpu.async_copy(cls_tab.at[icls.at[i]], rowb[b], gsa[b])
        return pltpu.async_copy(rel_tab.at[irel.at[i - NCLS]], rowb[b], gsa[b])

    def retire_a(i, h):
        b = i % 2
        h.wait()
        if i < NCLS:
            _accumulate_stats(rowb[b], i, stats_v)
            dst = cls_out.at[i, pl.ds(base, RPT)]
        else:
            dst = rel_out.at[i - NCLS, pl.ds(base, RPT)]
        return pltpu.async_copy(rowb[b], dst, wsa[b])

    def issue_b(i):
        b = i % 2
        return pltpu.async_copy(rad_tab.at[ihi.at[i]], radb[b], gsb[b])

    def retire_b(i, h):
        b = i % 2
        h.wait()
        _compact_rad(radb[b], icls, i, radc[b])
        return pltpu.async_copy(radc[b], rad_out.at[i, 0, pl.ds(base, RPT)],
                                wsb[b])

    ha = [None] * NA
    hb = [None] * NB
    wa = [None] * NA
    wb = [None] * NB
    ha[0] = issue_a(0)
    hb[0] = issue_b(0)
    for i in range(NA):
        if i + 1 < NA:
            if i - 1 >= 0:
                wa[i - 1].wait()
            ha[i + 1] = issue_a(i + 1)
        if i + 1 < NB:
            if i - 1 >= 0:
                wb[i - 1].wait()
            hb[i + 1] = issue_b(i + 1)
        wa[i] = retire_a(i, ha[i])
        if i < NB:
            wb[i] = retire_b(i, hb[i])
    wa[NA - 2].wait()
    wa[NA - 1].wait()
    wb[NB - 2].wait()
    wb[NB - 1].wait()
    pltpu.sync_copy(stats_v, stats_out.at[wid])


@functools.cache
def _sc_gather():
    return functools.partial(
        pl.kernel,
        out_type=[
            jax.ShapeDtypeStruct((NCLS, B, D), jnp.float32),
            jax.ShapeDtypeStruct((NREL, B, D), jnp.float32),
            jax.ShapeDtypeStruct((NCLS, 1, B), jnp.float32),
            jax.ShapeDtypeStruct((NW, 2 * NCLS, D), jnp.float32),
        ],
        scratch_types=[
            pltpu.VMEM((NCLS, RPT), jnp.int32),
            pltpu.VMEM((NCLS, RPT), jnp.int32),
            pltpu.VMEM((NREL, RPT), jnp.int32),
            pltpu.VMEM((RPT, D), jnp.float32),
            pltpu.VMEM((RPT, D), jnp.float32),
            pltpu.VMEM((RPT, 16), jnp.float32),
            pltpu.VMEM((RPT, 16), jnp.float32),
            pltpu.VMEM((RPT,), jnp.float32),
            pltpu.VMEM((RPT,), jnp.float32),
            pltpu.VMEM((2 * NCLS, D), jnp.float32),
        ] + [pltpu.SemaphoreType.DMA] * 9,
        **_SC_PARAMS,
    )(_sc_gather_body)


def _rsqrt_nr(x):
    i = lax.bitcast_convert_type(x, jnp.int32)
    i = 0x5F3759DF - lax.shift_right_logical(i, 1)
    y = lax.bitcast_convert_type(i, jnp.float32)
    for _ in range(3):
        y = y * (1.5 - 0.5 * x * y * y)
    return y


def _sqrt_nr(x):
    return x * _rsqrt_nr(x)


def _relu(x):
    return jnp.maximum(x, 0.0)


def _sc_loss_body(cls_rows, rel_rows, rad, scale, shift, out,
                  ba, bb, bc, ss_v, sh_v, rad_v, res_v, sem0, sem1, sem2):
    wid = lax.axis_index("s") * NC + lax.axis_index("c")
    base = wid * RPT
    iota = _iota16()

    pltpu.sync_copy(scale, ss_v)
    pltpu.sync_copy(shift, sh_v)
    pltpu.sync_copy(rad.at[:, 0, pl.ds(base, RPT)], rad_v)

    def fetch(buf, src, g, sem):
        return pltpu.async_copy(src.at[g, pl.ds(base, RPT)], buf, sem)

    def consts(g):
        s = [ss_v[g, pl.ds(c * 16, 16)] for c in range(4)]
        m = [sh_v[g, pl.ds(c * 16, 16)] for c in range(4)]
        return s, m

    rots = [lax.bitwise_and(iota + s, 15) for s in (8, 4, 2, 1)]

    def hsum16(v):
        for rot in rots:
            v = v + v.at[rot].get(mode="promise_in_bounds")
        return v

    def rowsum_merge(rowsums, k, n2):
        return jnp.where(iota == k, hsum16(n2), rowsums)

    def rad_at(g, grp):
        return jnp.abs(rad_v[g, pl.ds(grp * 16, 16)])

    acc = jnp.zeros((16,), jnp.float32)

    h1 = fetch(ba, cls_rows, 0, sem0)
    h2 = fetch(bb, cls_rows, 1, sem1)
    h3_pre = fetch(bc, cls_rows, 4, sem2)
    h1.wait()
    h2.wait()
    sa, ma = consts(0)
    sb, mb = consts(1)
    dm = [ma[c] - mb[c] for c in range(4)]

    def g0(grp, acc):
        rs = jnp.zeros((16,), jnp.float32)
        for k in range(16):
            r = grp * 16 + k
            n2 = None
            for c in range(4):
                sl = pl.ds(c * 16, 16)
                d = (ba[r, sl] * sa[c] - bb[r, sl] * sb[c]) - dm[c]
                t = d * d
                n2 = t if n2 is None else n2 + t
            rs = rowsum_merge(rs, k, n2)
        dst = _sqrt_nr(rs)
        return acc + _relu(dst + rad_at(0, grp) - rad_at(1, grp) - MARGIN)

    acc = lax.fori_loop(0, NGRP, g0, acc, unroll=2)

    h1 = fetch(ba, cls_rows, 2, sem0)
    h2 = fetch(bb, cls_rows, 3, sem1)
    h1.wait()
    h2.wait()
    h3_pre.wait()
    sa, ma = consts(2)
    sb, mb = consts(3)
    se, me = consts(4)

    def g1(grp, acc):
        rs1 = jnp.zeros((16,), jnp.float32)
        rs2 = jnp.zeros((16,), jnp.float32)
        rs3 = jnp.zeros((16,), jnp.float32)
        for k in range(16):
            r = grp * 16 + k
            n21 = n22 = n23 = None
            for c in range(4):
                sl = pl.ds(c * 16, 16)
                av = ba[r, sl] * sa[c] - ma[c]
                bv = bb[r, sl] * sb[c] - mb[c]
                ev = bc[r, sl] * se[c] - me[c]
                d1 = av - bv
                d2 = ev - av
                d3 = ev - bv
                t1 = d1 * d1
                t2 = d2 * d2
                t3 = d3 * d3
                n21 = t1 if n21 is None else n21 + t1
                n22 = t2 if n22 is None else n22 + t2
                n23 = t3 if n23 is None else n23 + t3
            rs1 = rowsum_merge(rs1, k, n21)
            rs2 = rowsum_merge(rs2, k, n22)
            rs3 = rowsum_merge(rs3, k, n23)
        ra = rad_at(2, grp)
        rb = rad_at(3, grp)
        t = _relu(_sqrt_nr(rs1) - (ra + rb) - MARGIN)
        t += _relu(_sqrt_nr(rs2) - ra - MARGIN)
        t += _relu(_sqrt_nr(rs3) - rb - MARGIN)
        return acc + t

    acc = lax.fori_loop(0, NGRP, g1, acc, unroll=2)

    h1 = fetch(ba, cls_rows, 5, sem0)
    h2 = fetch(bb, cls_rows, 6, sem1)
    h3 = fetch(bc, rel_rows, 0, sem2)
    h1.wait()
    h2.wait()
    h3.wait()
    sa, ma = consts(5)
    sb, mb = consts(6)
    dm = [ma[c] - mb[c] for c in range(4)]

    def g2(grp, acc):
        rs = jnp.zeros((16,), jnp.float32)
        for k in range(16):
            r = grp * 16 + k
            n2 = None
            for c in range(4):
                sl = pl.ds(c * 16, 16)
                d = (ba[r, sl] * sa[c] - bb[r, sl] * sb[c]) + bc[r, sl] - dm[c]
                t = d * d
                n2 = t if n2 is None else n2 + t
            rs = rowsum_merge(rs, k, n2)
        dst = _sqrt_nr(rs)
        ra = rad_at(5, grp)
        rb = rad_at(6, grp)
        t = _relu(dst + ra - rb - MARGIN)
        t += _relu(ra + rb - dst + MARGIN)
        return acc + t

    acc = lax.fori_loop(0, NGRP, g2, acc, unroll=2)

    h1 = fetch(ba, cls_rows, 7, sem0)
    h2 = fetch(bb, cls_rows, 8, sem1)
    h3 = fetch(bc, rel_rows, 1, sem2)
    h1.wait()
    h2.wait()
    h3.wait()
    sa, ma = consts(7)
    sb, mb = consts(8)
    dm = [ma[c] - mb[c] for c in range(4)]

    def g3(grp, acc):
        rs = jnp.zeros((16,), jnp.float32)
        for k in range(16):
            r = grp * 16 + k
            n2 = None
            for c in range(4):
                sl = pl.ds(c * 16, 16)
                d = (ba[r, sl] * sa[c] - bb[r, sl] * sb[c]) - bc[r, sl] - dm[c]
                t = d * d
                n2 = t if n2 is None else n2 + t
            rs = rowsum_merge(rs, k, n2)
        euc = _sqrt_nr(rs)
        ra = rad_at(7, grp)
        rb = rad_at(8, grp)
        return acc + _relu(euc - ra - rb - MARGIN)

    acc = lax.fori_loop(0, NGRP, g3, acc, unroll=2)

    res_v[...] = acc
    pltpu.sync_copy(res_v, out.at[wid])


@functools.cache
def _sc_loss():
    return functools.partial(
        pl.kernel,
        out_type=jax.ShapeDtypeStruct((NW, 16), jnp.float32),
        scratch_types=[
            pltpu.VMEM((RPT, D), jnp.float32),
            pltpu.VMEM((RPT, D), jnp.float32),
            pltpu.VMEM((RPT, D), jnp.float32),
            pltpu.VMEM((NCLS, D), jnp.float32),
            pltpu.VMEM((NCLS, D), jnp.float32),
            pltpu.VMEM((NCLS, RPT), jnp.float32),
            pltpu.VMEM((16,), jnp.float32),
            pltpu.SemaphoreType.DMA,
            pltpu.SemaphoreType.DMA,
            pltpu.SemaphoreType.DMA,
        ],
        **_SC_PARAMS,
    )(_sc_loss_body)


def kernel(nf0, nf1, nf2, nf3, class_embed_w, class_rad_w, rel_embed_w,
           bn_gamma, bn_beta):
    del bn_beta
    cls_idx = jnp.stack([
        nf0[:, 0], nf0[:, 1],
        nf1[:, 0], nf1[:, 1], nf1[:, 2],
        nf2[:, 0], nf2[:, 2],
        nf3[:, 1], nf3[:, 2],
    ]).astype(jnp.int32).reshape(NCLS, 1, B)
    rel_idx = jnp.stack([nf2[:, 1], nf3[:, 0]]).astype(jnp.int32)

    cls_rows, rel_rows, rad, stats = _sc_gather()(
        cls_idx, rel_idx.reshape(NREL, 1, B),
        class_embed_w, class_rad_w.reshape(-1, 16), rel_embed_w)

    sums = jnp.sum(stats[:, :NCLS, :], axis=0)
    sumsqs = jnp.sum(stats[:, NCLS:, :], axis=0)
    inv_b = 1.0 / B
    mean = sums * inv_b
    var = sumsqs * inv_b - mean * mean
    scale = bn_gamma[None, :] / jnp.sqrt(var + EPS)
    shift = scale * mean

    parts = _sc_loss()(cls_rows, rel_rows, rad, scale, shift)
    return jnp.sum(parts) * inv_b


# speedup vs baseline: 1.0761x; 1.0013x over previous
import functools

import jax
import jax.numpy as jnp
from jax import lax
from jax.experimental import pallas as pl
from jax.experimental.pallas import tpu as pltpu
from jax.experimental.pallas import tpu_sc as plsc

B = 16384
D = 64
MARGIN = 0.1
EPS = 1e-5

NC = 2
NS = 16
NW = NC * NS
RPT = B // NW
NGRP = RPT // 16

NCLS = 9
NREL = 2

_SC_PARAMS = dict(
    mesh=plsc.VectorSubcoreMesh(core_axis_name="c", subcore_axis_name="s"),
    compiler_params=pltpu.CompilerParams(use_tc_tiling_on_sc=False),
)


def _iota16():
    return lax.broadcasted_iota(jnp.int32, (16,), 0)


def _accumulate_stats(buf, g, stats_v):
    def step(r, carry):
        out = []
        outsq = []
        for c in range(4):
            v = buf[r, pl.ds(c * 16, 16)]
            out.append(carry[c] + v)
            outsq.append(carry[4 + c] + v * v)
        return tuple(out) + tuple(outsq)

    init = tuple(jnp.zeros((16,), jnp.float32) for _ in range(8))
    accs = lax.fori_loop(0, RPT, step, init, unroll=8)
    for c in range(4):
        stats_v[g, pl.ds(c * 16, 16)] = accs[c]
        stats_v[NCLS + g, pl.ds(c * 16, 16)] = accs[4 + c]


def _compact_rad(radb, icls, g, rad_cv):
    iota = _iota16()

    def step(grp, _):
        lo16 = lax.bitwise_and(icls[g, pl.ds(grp * 16, 16)], 15)
        cvec = jnp.zeros((16,), jnp.float32)
        for k in range(16):
            row = radb[grp * 16 + k, :]
            sel = row.at[lo16].get(mode="promise_in_bounds")
            cvec = jnp.where(iota == k, sel, cvec)
        rad_cv[pl.ds(grp * 16, 16)] = cvec
        return 0

    lax.fori_loop(0, NGRP, step, 0, unroll=2)


def _sc_gather_body(cls_idx, rel_idx, cls_tab, rad_tab, rel_tab,
                    cls_out, rel_out, rad_out, stats_out,
                    icls, ihi, irel, rowb0, rowb1, radb0, radb1,
                    radc0, radc1, stats_v,
                    isem, gsa0, gsa1, gsb0, gsb1, wsa0, wsa1, wsb0, wsb1):
    wid = lax.axis_index("s") * NC + lax.axis_index("c")
    base = wid * RPT
    rowb = [rowb0, rowb1]
    radb = [radb0, radb1]
    radc = [radc0, radc1]
    gsa = [gsa0, gsa1]
    gsb = [gsb0, gsb1]
    wsa = [wsa0, wsa1]
    wsb = [wsb0, wsb1]

    h1 = pltpu.async_copy(cls_idx.at[:, 0, pl.ds(base, RPT)], icls, isem)
    h2 = pltpu.async_copy(rel_idx.at[:, 0, pl.ds(base, RPT)], irel, isem)
    h1.wait()
    h2.wait()
    for g in range(NCLS):
        for grp in range(NGRP):
            sl = pl.ds(grp * 16, 16)
            ihi[g, sl] = lax.shift_right_logical(icls[g, sl], 4)

    NA = NCLS + NREL
    NB = NCLS

    def issue_a(i):
        b = i % 2
        if i < NCLS:
            return pltpu.async_copy(cls_tab.at[icls.at[i]], rowb[b], gsa[b])
        return pltpu.async_copy(rel_tab.at[irel.at[i - NCLS]], rowb[b], gsa[b])

    def retire_a(i, h):
        b = i % 2
        h.wait()
        if i < NCLS:
            _accumulate_stats(rowb[b], i, stats_v)
            dst = cls_out.at[i, pl.ds(base, RPT)]
        else:
            dst = rel_out.at[i - NCLS, pl.ds(base, RPT)]
        return pltpu.async_copy(rowb[b], dst, wsa[b])

    def issue_b(i):
        b = i % 2
        return pltpu.async_copy(rad_tab.at[ihi.at[i]], radb[b], gsb[b])

    def retire_b(i, h):
        b = i % 2
        h.wait()
        _compact_rad(radb[b], icls, i, radc[b])
        return pltpu.async_copy(radc[b], rad_out.at[i, 0, pl.ds(base, RPT)],
                                wsb[b])

    ha = [None] * NA
    hb = [None] * NB
    wa = [None] * NA
    wb = [None] * NB
    ha[0] = issue_a(0)
    hb[0] = issue_b(0)
    for i in range(NA):
        if i + 1 < NA:
            if i - 1 >= 0:
                wa[i - 1].wait()
            ha[i + 1] = issue_a(i + 1)
        if i + 1 < NB:
            if i - 1 >= 0:
                wb[i - 1].wait()
            hb[i + 1] = issue_b(i + 1)
        wa[i] = retire_a(i, ha[i])
        if i < NB:
            wb[i] = retire_b(i, hb[i])
    wa[NA - 2].wait()
    wa[NA - 1].wait()
    wb[NB - 2].wait()
    wb[NB - 1].wait()
    pltpu.sync_copy(stats_v, stats_out.at[wid])


@functools.cache
def _sc_gather():
    return functools.partial(
        pl.kernel,
        out_type=[
            jax.ShapeDtypeStruct((NCLS, B, D), jnp.float32),
            jax.ShapeDtypeStruct((NREL, B, D), jnp.float32),
            jax.ShapeDtypeStruct((NCLS, 1, B), jnp.float32),
            jax.ShapeDtypeStruct((NW, 2 * NCLS, D), jnp.float32),
        ],
        scratch_types=[
            pltpu.VMEM((NCLS, RPT), jnp.int32),
            pltpu.VMEM((NCLS, RPT), jnp.int32),
            pltpu.VMEM((NREL, RPT), jnp.int32),
            pltpu.VMEM((RPT, D), jnp.float32),
            pltpu.VMEM((RPT, D), jnp.float32),
            pltpu.VMEM((RPT, 16), jnp.float32),
            pltpu.VMEM((RPT, 16), jnp.float32),
            pltpu.VMEM((RPT,), jnp.float32),
            pltpu.VMEM((RPT,), jnp.float32),
            pltpu.VMEM((2 * NCLS, D), jnp.float32),
        ] + [pltpu.SemaphoreType.DMA] * 9,
        **_SC_PARAMS,
    )(_sc_gather_body)


def _rsqrt_nr(x):
    i = lax.bitcast_convert_type(x, jnp.int32)
    i = 0x5F3759DF - lax.shift_right_logical(i, 1)
    y = lax.bitcast_convert_type(i, jnp.float32)
    for _ in range(3):
        y = y * (1.5 - 0.5 * x * y * y)
    return y


def _sqrt_nr(x):
    return x * _rsqrt_nr(x)


def _relu(x):
    return jnp.maximum(x, 0.0)


def _sc_loss_body(cls_rows, rel_rows, rad, scale, shift, out,
                  ba, bb, bc, ss_v, sh_v, rad_v, res_v, sem0, sem1, sem2):
    wid = lax.axis_index("s") * NC + lax.axis_index("c")
    base = wid * RPT
    iota = _iota16()

    pltpu.sync_copy(scale, ss_v)
    pltpu.sync_copy(shift, sh_v)
    pltpu.sync_copy(rad.at[:, 0, pl.ds(base, RPT)], rad_v)

    def fetch(buf, src, g, sem):
        return pltpu.async_copy(src.at[g, pl.ds(base, RPT)], buf, sem)

    def consts(g):
        s = [ss_v[g, pl.ds(c * 16, 16)] for c in range(4)]
        m = [sh_v[g, pl.ds(c * 16, 16)] for c in range(4)]
        return s, m

    rots = [lax.bitwise_and(iota + s, 15) for s in (8, 4, 2, 1)]

    def hsum16(v):
        for rot in rots:
            v = v + v.at[rot].get(mode="promise_in_bounds")
        return v

    def rowsum_merge(rowsums, k, n2):
        return jnp.where(iota == k, hsum16(n2), rowsums)

    def rad_at(g, grp):
        return jnp.abs(rad_v[g, pl.ds(grp * 16, 16)])

    acc = jnp.zeros((16,), jnp.float32)

    h1 = fetch(ba, cls_rows, 0, sem0)
    h2 = fetch(bb, cls_rows, 1, sem1)
    h3_pre = fetch(bc, cls_rows, 4, sem2)
    h1.wait()
    h2.wait()
    sa, ma = consts(0)
    sb, mb = consts(1)
    dm = [ma[c] - mb[c] for c in range(4)]

    def g0(grp, acc):
        rs = jnp.zeros((16,), jnp.float32)
        for k in range(16):
            r = grp * 16 + k
            n2 = None
            for c in range(4):
                sl = pl.ds(c * 16, 16)
                d = (ba[r, sl] * sa[c] - bb[r, sl] * sb[c]) - dm[c]
                t = d * d
                n2 = t if n2 is None else n2 + t
            rs = rowsum_merge(rs, k, n2)
        dst = _sqrt_nr(rs)
        return acc + _relu(dst + rad_at(0, grp) - rad_at(1, grp) - MARGIN)

    acc = lax.fori_loop(0, NGRP, g0, acc, unroll=2)

    h1 = fetch(ba, cls_rows, 2, sem0)
    h2 = fetch(bb, cls_rows, 3, sem1)
    h1.wait()
    h2.wait()
    h3_pre.wait()
    sa, ma = consts(2)
    sb, mb = consts(3)
    se, me = consts(4)

    def g1(grp, acc):
        rs1 = jnp.zeros((16,), jnp.float32)
        rs2 = jnp.zeros((16,), jnp.float32)
        rs3 = jnp.zeros((16,), jnp.float32)
        for k in range(16):
            r = grp * 16 + k
            n21 = n22 = n23 = None
            for c in range(4):
                sl = pl.ds(c * 16, 16)
                av = ba[r, sl] * sa[c] - ma[c]
                bv = bb[r, sl] * sb[c] - mb[c]
                ev = bc[r, sl] * se[c] - me[c]
                d1 = av - bv
                d2 = ev - av
                d3 = ev - bv
                t1 = d1 * d1
                t2 = d2 * d2
                t3 = d3 * d3
                n21 = t1 if n21 is None else n21 + t1
                n22 = t2 if n22 is None else n22 + t2
                n23 = t3 if n23 is None else n23 + t3
            rs1 = rowsum_merge(rs1, k, n21)
            rs2 = rowsum_merge(rs2, k, n22)
            rs3 = rowsum_merge(rs3, k, n23)
        ra = rad_at(2, grp)
        rb = rad_at(3, grp)
        t = _relu(_sqrt_nr(rs1) - (ra + rb) - MARGIN)
        t += _relu(_sqrt_nr(rs2) - ra - MARGIN)
        t += _relu(_sqrt_nr(rs3) - rb - MARGIN)
        return acc + t

    acc = lax.fori_loop(0, NGRP, g1, acc, unroll=2)

    h1 = fetch(ba, cls_rows, 5, sem0)
    h2 = fetch(bb, cls_rows, 6, sem1)
    h3 = fetch(bc, rel_rows, 0, sem2)
    h1.wait()
    h2.wait()
    h3.wait()
    sa, ma = consts(5)
    sb, mb = consts(6)
    dm = [ma[c] - mb[c] for c in range(4)]

    def g2(grp, acc):
        rs = jnp.zeros((16,), jnp.float32)
        for k in range(16):
            r = grp * 16 + k
            n2 = None
            for c in range(4):
                sl = pl.ds(c * 16, 16)
                d = (ba[r, sl] * sa[c] - bb[r, sl] * sb[c]) + bc[r, sl] - dm[c]
                t = d * d
                n2 = t if n2 is None else n2 + t
            rs = rowsum_merge(rs, k, n2)
        dst = _sqrt_nr(rs)
        ra = rad_at(5, grp)
        rb = rad_at(6, grp)
        t = _relu(dst + ra - rb - MARGIN)
        t += _relu(ra + rb - dst + MARGIN)
        return acc + t

    acc = lax.fori_loop(0, NGRP, g2, acc, unroll=2)

    h1 = fetch(ba, cls_rows, 7, sem0)
    h2 = fetch(bb, cls_rows, 8, sem1)
    h3 = fetch(bc, rel_rows, 1, sem2)
    h1.wait()
    h2.wait()
    h3.wait()
    sa, ma = consts(7)
    sb, mb = consts(8)
    dm = [ma[c] - mb[c] for c in range(4)]

    def g3(grp, acc):
        rs = jnp.zeros((16,), jnp.float32)
        for k in range(16):
            r = grp * 16 + k
            n2 = None
            for c in range(4):
                sl = pl.ds(c * 16, 16)
                d = (ba[r, sl] * sa[c] - bb[r, sl] * sb[c]) - bc[r, sl] - dm[c]
                t = d * d
                n2 = t if n2 is None else n2 + t
            rs = rowsum_merge(rs, k, n2)
        euc = _sqrt_nr(rs)
        ra = rad_at(7, grp)
        rb = rad_at(8, grp)
        return acc + _relu(euc - ra - rb - MARGIN)

    acc = lax.fori_loop(0, NGRP, g3, acc, unroll=2)

    res_v[...] = acc
    pltpu.sync_copy(res_v, out.at[wid])


@functools.cache
def _sc_loss():
    return functools.partial(
        pl.kernel,
        out_type=jax.ShapeDtypeStruct((NW, 16), jnp.float32),
        scratch_types=[
            pltpu.VMEM((RPT, D), jnp.float32),
            pltpu.VMEM((RPT, D), jnp.float32),
            pltpu.VMEM((RPT, D), jnp.float32),
            pltpu.VMEM((NCLS, D), jnp.float32),
            pltpu.VMEM((NCLS, D), jnp.float32),
            pltpu.VMEM((NCLS, RPT), jnp.float32),
            pltpu.VMEM((16,), jnp.float32),
            pltpu.SemaphoreType.DMA,
            pltpu.SemaphoreType.DMA,
            pltpu.SemaphoreType.DMA,
        ],
        **_SC_PARAMS,
    )(_sc_loss_body)


def kernel(nf0, nf1, nf2, nf3, class_embed_w, class_rad_w, rel_embed_w,
           bn_gamma, bn_beta):
    del bn_beta
    cls_idx = jnp.stack([
        nf0[:, 0], nf0[:, 1],
        nf1[:, 0], nf1[:, 1], nf1[:, 2],
        nf2[:, 0], nf2[:, 2],
        nf3[:, 1], nf3[:, 2],
    ]).astype(jnp.int32).reshape(NCLS, 1, B)
    rel_idx = jnp.stack([nf2[:, 1], nf3[:, 0]]).astype(jnp.int32)

    cls_rows, rel_rows, rad, stats = _sc_gather()(
        cls_idx, rel_idx.reshape(NREL, 1, B),
        class_embed_w, class_rad_w.reshape(-1, 16), rel_embed_w)

    sums = jnp.sum(stats[:, :NCLS, :], axis=0)
    sumsqs = jnp.sum(stats[:, NCLS:, :], axis=0)
    inv_b = 1.0 / B
    mean = sums * inv_b
    var = sumsqs * inv_b - mean * mean
    scale = bn_gamma[None, :] / jnp.sqrt(var + EPS)
    shift = scale * mean

    parts = _sc_loss()(cls_rows, rel_rows, rad, scale, shift)
    return jnp.sum(parts) * inv_b
